# Initial kernel scaffold; baseline (speedup 1.0000x reference)
import jax
import jax.numpy as jnp
from jax.experimental import pallas as pl


def kernel(x, offsets):
    raise NotImplementedError("write your pallas kernel here")


# speedup vs baseline: 2.0297x; 2.0297x over previous
import functools

import jax
import jax.numpy as jnp
from jax import lax
from jax.experimental import pallas as pl
from jax.experimental.pallas import tpu as pltpu
from jax.experimental.pallas import tpu_sc as plsc

TOTAL = 8192
LANES = 16


def _build(B, N, C, NC, NS):
    NW = NC * NS
    rows_w = TOTAL // NW
    CH = 64
    NCH = rows_w // CH
    GPC = CH // LANES

    mesh = plsc.VectorSubcoreMesh(core_axis_name="c", subcore_axis_name="s")

    @functools.partial(
        pl.kernel,
        out_type=jax.ShapeDtypeStruct((TOTAL, C), jnp.float32),
        mesh=mesh,
        compiler_params=pltpu.CompilerParams(needs_layout_passes=False),
        scratch_types=[
            pltpu.VMEM((LANES,), jnp.int32),
            pltpu.VMEM((NCH, CH), jnp.int32),
            pltpu.VMEM((2, CH, C), jnp.float32),
            pltpu.SemaphoreType.DMA,
            pltpu.SemaphoreType.DMA,
            pltpu.SemaphoreType.DMA,
            pltpu.SemaphoreType.DMA,
        ],
    )
    def k(x_hbm, offs_hbm, out_hbm, offs_v, idx_v, bufs, gs0, gs1, ss0, ss1):
        wid = lax.axis_index("s") * NC + lax.axis_index("c")
        base = wid * rows_w

        pltpu.sync_copy(offs_hbm, offs_v)

        bounds = [
            plsc.load_gather(offs_v, [jnp.full((LANES,), j, jnp.int32)])
            for j in range(1, B + 1)
        ]

        for g in range(rows_w // LANES):
            tok = base + g * LANES + lax.iota(jnp.int32, 16)
            b = jnp.zeros((LANES,), jnp.int32)
            for bd in bounds:
                b = b + (bd <= tok).astype(jnp.int32)
            start = plsc.load_gather(offs_v, [b])
            row = b * N + (tok - start)
            idx_v[g // GPC, pl.ds((g % GPC) * LANES, LANES)] = row

        gsems = [gs0, gs1]
        ssems = [ss0, ss1]
        copies = [None, None]
        for c in range(NCH):
            s = c % 2
            if copies[s] is not None:
                copies[s].wait()
            pltpu.async_copy(x_hbm.at[idx_v.at[c]], bufs.at[s], gsems[s]).wait()
            copies[s] = pltpu.async_copy(
                bufs.at[s], out_hbm.at[pl.ds(base + c * CH, CH)], ssems[s]
            )
        for s in range(2):
            if copies[s] is not None:
                copies[s].wait()

    return k


def kernel(x, offsets):
    B, N, C = x.shape
    info = plsc.get_sparse_core_info()
    xflat = x.reshape(B * N, C)
    offs_pad = jnp.zeros((LANES,), jnp.int32).at[: B + 1].set(offsets)
    k = _build(B, N, C, info.num_cores, info.num_subcores)
    return k(xflat, offs_pad)


# speedup vs baseline: 2.0917x; 1.0306x over previous
import functools

import jax
import jax.numpy as jnp
from jax import lax
from jax.experimental import pallas as pl
from jax.experimental.pallas import tpu as pltpu
from jax.experimental.pallas import tpu_sc as plsc

TOTAL = 8192
LANES = 16


def _build(B, N, C, NC, NS):
    NW = NC * NS
    rows_w = TOTAL // NW
    CH = 32
    NCH = rows_w // CH
    GPC = CH // LANES
    NB = 4

    mesh = plsc.VectorSubcoreMesh(core_axis_name="c", subcore_axis_name="s")

    @functools.partial(
        pl.kernel,
        out_type=jax.ShapeDtypeStruct((TOTAL, C), jnp.float32),
        mesh=mesh,
        compiler_params=pltpu.CompilerParams(needs_layout_passes=False),
        scratch_types=[
            pltpu.VMEM((LANES,), jnp.int32),
            pltpu.VMEM((NCH, CH), jnp.int32),
            pltpu.VMEM((NB, CH, C), jnp.float32),
            [pltpu.SemaphoreType.DMA] * NB,
            [pltpu.SemaphoreType.DMA] * NB,
        ],
    )
    def k(x_hbm, offs_hbm, out_hbm, offs_v, idx_v, bufs, gsems, ssems):
        wid = lax.axis_index("s") * NC + lax.axis_index("c")
        base = wid * rows_w

        pltpu.sync_copy(offs_hbm, offs_v)

        bounds = [
            plsc.load_gather(offs_v, [jnp.full((LANES,), j, jnp.int32)])
            for j in range(1, B + 1)
        ]

        gath = [None] * NB
        stor = [None] * NB

        def fire_gather(c):
            s = c % NB
            if stor[s] is not None:
                stor[s].wait()
            gath[s] = pltpu.async_copy(x_hbm.at[idx_v.at[c]], bufs.at[s], gsems[s])

        for g in range(rows_w // LANES):
            tok = base + g * LANES + lax.iota(jnp.int32, 16)
            b = jnp.zeros((LANES,), jnp.int32)
            for bd in bounds:
                b = b + (bd <= tok).astype(jnp.int32)
            start = plsc.load_gather(offs_v, [b])
            row = b * N + (tok - start)
            idx_v[g // GPC, pl.ds((g % GPC) * LANES, LANES)] = row
            c = g // GPC
            if g % GPC == GPC - 1 and c < NB:
                fire_gather(c)

        for c in range(NCH):
            s = c % NB
            gath[s].wait()
            stor[s] = pltpu.async_copy(
                bufs.at[s], out_hbm.at[pl.ds(base + c * CH, CH)], ssems[s]
            )
            if c + NB < NCH:
                fire_gather(c + NB)
        for s in range(NB):
            if stor[s] is not None:
                stor[s].wait()

    return k


def kernel(x, offsets):
    B, N, C = x.shape
    info = plsc.get_sparse_core_info()
    xflat = x.reshape(B * N, C)
    offs_pad = jnp.zeros((LANES,), jnp.int32).at[: B + 1].set(offsets)
    k = _build(B, N, C, info.num_cores, info.num_subcores)
    return k(xflat, offs_pad)
